# baseline (speedup 1.0000x reference)
import functools

import jax
import jax.numpy as jnp
from jax import lax
from jax.experimental import pallas as pl
from jax.experimental.pallas import tpu as pltpu
from jax.experimental.pallas import tpu_sc as plsc

B, S, D = 4, 8192, 1024
NC, NS, L = 2, 16, 16
NW = NC * NS
ROWS_W = S // NW
CH = 16
CHW = CH * D
N_CHUNKS = ROWS_W // CH

_mesh = plsc.VectorSubcoreMesh(
    core_axis_name="c", subcore_axis_name="s", num_cores=NC, num_subcores=NS
)


def _add_chunk(o_ref, t_ref):
    def body(i, _):
        base = i * (4 * L)
        for j in range(4):
            sl = pl.ds(base + j * L, L)
            plsc.addupdate(o_ref.at[sl], t_ref[sl])
        return 0

    lax.fori_loop(0, CHW // (4 * L), body, 0, unroll=False)


@functools.partial(
    pl.kernel,
    out_type=jax.ShapeDtypeStruct((B * S * D,), jnp.float32),
    mesh=_mesh,
    scratch_types=[
        pltpu.VMEM((CHW,), jnp.float32),
        pltpu.VMEM((CHW,), jnp.float32),
        pltpu.VMEM((CHW,), jnp.float32),
        pltpu.SemaphoreType.DMA,
        pltpu.SemaphoreType.DMA,
        pltpu.SemaphoreType.DMA,
        pltpu.SemaphoreType.DMA,
        pltpu.SemaphoreType.DMA,
    ],
)
def _pos_add_sc(x_hbm, pos_hbm, out_hbm, t_ref, o0, o1, st, si0, si1, so0, so1):
    wid = lax.axis_index("s") * NC + lax.axis_index("c")
    base = wid * (ROWS_W * D)

    def chunk(c, _):
        off = base + c * CHW
        tin = pltpu.make_async_copy(pos_hbm.at[pl.ds(off, CHW)], t_ref, st)
        tin.start()
        in0 = pltpu.make_async_copy(x_hbm.at[pl.ds(0 * S * D + off, CHW)], o0, si0)
        in1 = pltpu.make_async_copy(x_hbm.at[pl.ds(1 * S * D + off, CHW)], o1, si1)
        in2 = pltpu.make_async_copy(x_hbm.at[pl.ds(2 * S * D + off, CHW)], o0, si0)
        in3 = pltpu.make_async_copy(x_hbm.at[pl.ds(3 * S * D + off, CHW)], o1, si1)
        out0 = pltpu.make_async_copy(o0, out_hbm.at[pl.ds(0 * S * D + off, CHW)], so0)
        out1 = pltpu.make_async_copy(o1, out_hbm.at[pl.ds(1 * S * D + off, CHW)], so1)
        out2 = pltpu.make_async_copy(o0, out_hbm.at[pl.ds(2 * S * D + off, CHW)], so0)
        out3 = pltpu.make_async_copy(o1, out_hbm.at[pl.ds(3 * S * D + off, CHW)], so1)

        in0.start()
        in1.start()
        tin.wait()
        in0.wait()
        _add_chunk(o0, t_ref)
        out0.start()
        in1.wait()
        _add_chunk(o1, t_ref)
        out1.start()
        out0.wait()
        in2.start()
        out1.wait()
        in3.start()
        in2.wait()
        _add_chunk(o0, t_ref)
        out2.start()
        in3.wait()
        _add_chunk(o1, t_ref)
        out3.start()
        out2.wait()
        out3.wait()
        return 0

    lax.fori_loop(0, N_CHUNKS, chunk, 0, unroll=False)


def kernel(x, pos_table):
    xf = x.reshape(B * S * D)
    pf = pos_table[:S].reshape(S * D)
    out = _pos_add_sc(xf, pf)
    return out.reshape(B, S, D)
